# Initial kernel scaffold; baseline (speedup 1.0000x reference)
import jax
import jax.numpy as jnp
from jax.experimental import pallas as pl


def kernel(mk_xy, mk_depth, fullres_disps):
    raise NotImplementedError("write your pallas kernel here")


# speedup vs baseline: 4.4324x; 4.4324x over previous
import functools

import jax
import jax.numpy as jnp
from jax import lax
from jax.experimental import pallas as pl
from jax.experimental.pallas import tpu as pltpu
from jax.experimental.pallas import tpu_sc as plsc

B, V, K = 8, 2, 8192
H, W = 512, 512
HW = H * W
N = V * B * HW
NPLANES = V * B
KV = K // 16
SLAB = 64 * W
NSLAB = HW // 2 // SLAB

_mesh = plsc.VectorSubcoreMesh(core_axis_name="c", subcore_axis_name="s")

_GATHER_DNUMS = lax.GatherDimensionNumbers(
    offset_dims=(), collapsed_slice_dims=(0,), start_index_map=(0,)
)


def _dyn_gather(vec, idx):
    return lax.gather(
        vec,
        idx[:, None],
        dimension_numbers=_GATHER_DNUMS,
        slice_sizes=(1,),
        mode=lax.GatherScatterMode.PROMISE_IN_BOUNDS,
    )


_SCRATCH_TYPES = [
    pltpu.VMEM((K,), jnp.int32),
    pltpu.VMEM((K,), jnp.int32),
    pltpu.VMEM((K,), jnp.float32),
    pltpu.VMEM((K,), jnp.int32),
    pltpu.VMEM((K,), jnp.int32),
    pltpu.VMEM((K,), jnp.float32),
    pltpu.VMEM((SLAB,), jnp.float32),
]


def _fuse_body(x_hbm, y_hbm, z_hbm, in_hbm, out_hbm, xv, yv, zv, av, wv, vv, slab):
    cid = lax.axis_index("c")
    sid = lax.axis_index("s")
    p = sid
    b = lax.rem(p, B)
    v = lax.div(p, B)
    blk = (b * V + v) * K

    pltpu.sync_copy(x_hbm.at[pl.ds(blk, K)], xv)
    pltpu.sync_copy(y_hbm.at[pl.ds(blk, K)], yv)
    pltpu.sync_copy(z_hbm.at[pl.ds(blk, K)], zv)

    lane = lax.iota(jnp.int32, 16)

    def addr_body(i, carry):
        sl = pl.ds(i * 16, 16)
        l = yv[sl] * W + xv[sl]
        av[sl] = l
        loser = l != l
        for sft in range(1, 16):
            perm = lax.bitwise_and(lane + sft, 15)
            rot = _dyn_gather(l, perm)
            loser = loser | ((rot == l) & (lane < (16 - sft)))
        wv[sl] = jnp.where(loser, 0, 1)
        return carry

    lax.fori_loop(0, KV, addr_body, 0)

    for s in range(NSLAB):
        lo = (cid * (HW // 2)) + s * SLAB
        gbase = p * HW + lo
        pltpu.sync_copy(in_hbm.at[pl.ds(gbase, SLAB)], slab)

        def gather_body(i, carry, lo=lo):
            sl = pl.ds(i * 16, 16)
            l = av[sl] - lo
            m = (l >= 0) & (l < SLAB)
            off = jnp.where(m, l, 0)
            orig = plsc.load_gather(slab, [off], mask=m)
            vv[sl] = 0.5 * (1.0 / zv[sl]) + 0.5 * orig
            return carry

        lax.fori_loop(0, KV, gather_body, 0)

        def scatter_body(i, carry, lo=lo):
            sl = pl.ds(i * 16, 16)
            l = av[sl] - lo
            m = (l >= 0) & (l < SLAB) & (wv[sl] != 0)
            off = jnp.where(m, l, 0)
            plsc.store_scatter(slab, [off], vv[sl], mask=m)
            return carry

        lax.fori_loop(0, KV, scatter_body, 0)

        pltpu.sync_copy(slab, out_hbm.at[pl.ds(gbase, SLAB)])


_fuse = pl.kernel(
    _fuse_body,
    out_type=jax.ShapeDtypeStruct((N,), jnp.float32),
    mesh=_mesh,
    scratch_types=_SCRATCH_TYPES,
    compiler_params=pltpu.CompilerParams(needs_layout_passes=False),
)


def kernel(mk_xy, mk_depth, fullres_disps):
    x = mk_xy[..., 0].reshape(-1)
    y = mk_xy[..., 1].reshape(-1)
    z = mk_depth.reshape(-1)
    in_flat = fullres_disps.reshape(N)
    out = _fuse(x, y, z, in_flat)
    return out.reshape(V * B, 1, H, W)
